# Initial kernel scaffold; baseline (speedup 1.0000x reference)
import jax
import jax.numpy as jnp
from jax.experimental import pallas as pl


def kernel(x, w_ih_l0, w_hh_l0, b_ih_l0, b_hh_l0, w_ih_l1, w_hh_l1, b_ih_l1, b_hh_l1):
    raise NotImplementedError("write your pallas kernel here")


# speedup vs baseline: 10.7034x; 10.7034x over previous
import jax
import jax.numpy as jnp
from jax.experimental import pallas as pl
from jax.experimental.pallas import tpu as pltpu

B, T, D, H = 8, 2048, 256, 256
CHUNK = 256
NCHUNK = T // CHUNK


def _gru2_kernel(xt_ref, wih0_ref, whh0_ref, bih0_ref, bhh0_ref,
                 wih1_ref, whh1_ref, bih1_ref, bhh1_ref,
                 out_ref, gi_ref, h1buf_ref, h1_ref, h2_ref):
    @pl.when(pl.program_id(0) == 0)
    def _init():
        h1_ref[...] = jnp.zeros_like(h1_ref)
        h2_ref[...] = jnp.zeros_like(h2_ref)

    def cell(gi, gh, h):
        r = jax.nn.sigmoid(gi[:, :H] + gh[:, :H])
        z = jax.nn.sigmoid(gi[:, H:2 * H] + gh[:, H:2 * H])
        n = jnp.tanh(gi[:, 2 * H:] + r * gh[:, 2 * H:])
        return n + z * (h - n)

    gi_ref[...] = (jnp.dot(xt_ref[...], wih0_ref[...],
                           preferred_element_type=jnp.float32)
                   + bih0_ref[...])

    def step1(t, h):
        gi = gi_ref[pl.ds(t * B, B), :]
        gh = (jnp.dot(h, whh0_ref[...], preferred_element_type=jnp.float32)
              + bhh0_ref[...])
        hn = cell(gi, gh, h)
        h1buf_ref[pl.ds(t * B, B), :] = hn
        return hn

    h1_ref[...] = jax.lax.fori_loop(0, CHUNK, step1, h1_ref[...])

    gi_ref[...] = (jnp.dot(h1buf_ref[...], wih1_ref[...],
                           preferred_element_type=jnp.float32)
                   + bih1_ref[...])

    def step2(t, h):
        gi = gi_ref[pl.ds(t * B, B), :]
        gh = (jnp.dot(h, whh1_ref[...], preferred_element_type=jnp.float32)
              + bhh1_ref[...])
        hn = cell(gi, gh, h)
        out_ref[pl.ds(t * B, B), :] = hn
        return hn

    h2_ref[...] = jax.lax.fori_loop(0, CHUNK, step2, h2_ref[...])


def kernel(x, w_ih_l0, w_hh_l0, b_ih_l0, b_hh_l0,
           w_ih_l1, w_hh_l1, b_ih_l1, b_hh_l1):
    xt = jnp.swapaxes(x, 0, 1).reshape(T * B, D)

    full2d = lambda shape: pl.BlockSpec(shape, lambda i: (0, 0))
    out2d = pl.pallas_call(
        _gru2_kernel,
        grid=(NCHUNK,),
        in_specs=[
            pl.BlockSpec((CHUNK * B, D), lambda i: (i, 0)),
            full2d((D, 3 * H)), full2d((H, 3 * H)),
            full2d((1, 3 * H)), full2d((1, 3 * H)),
            full2d((H, 3 * H)), full2d((H, 3 * H)),
            full2d((1, 3 * H)), full2d((1, 3 * H)),
        ],
        out_specs=pl.BlockSpec((CHUNK * B, H), lambda i: (i, 0)),
        out_shape=jax.ShapeDtypeStruct((T * B, H), jnp.float32),
        scratch_shapes=[
            pltpu.VMEM((CHUNK * B, 3 * H), jnp.float32),
            pltpu.VMEM((CHUNK * B, H), jnp.float32),
            pltpu.VMEM((B, H), jnp.float32),
            pltpu.VMEM((B, H), jnp.float32),
        ],
        compiler_params=pltpu.CompilerParams(
            dimension_semantics=("arbitrary",),
        ),
    )(xt, w_ih_l0.T, w_hh_l0.T, b_ih_l0.reshape(1, -1), b_hh_l0.reshape(1, -1),
      w_ih_l1.T, w_hh_l1.T, b_ih_l1.reshape(1, -1), b_hh_l1.reshape(1, -1))

    return jnp.swapaxes(out2d.reshape(T, B, H), 0, 1)


# speedup vs baseline: 13.8169x; 1.2909x over previous
import jax
import jax.numpy as jnp
from jax.experimental import pallas as pl
from jax.experimental.pallas import tpu as pltpu

B, T, D, H = 8, 2048, 256, 256
CHUNK = 256
NCHUNK = T // CHUNK


def _cell(gi, gh, h):
    rz = jax.nn.sigmoid(gi[:, :2 * H] + gh[:, :2 * H])
    r = rz[:, :H]
    z = rz[:, H:]
    n = jnp.tanh(gi[:, 2 * H:] + r * gh[:, 2 * H:])
    return n + z * (h - n)


def _gru2_kernel(xt_ref, wih0_ref, w12_ref, b12_ref, whh1_ref, bhh1_ref,
                 bih0_ref, out_ref, gi_ref, h1_ref, h2_ref):
    @pl.when(pl.program_id(0) == 0)
    def _init():
        h1_ref[...] = jnp.zeros_like(h1_ref)
        h2_ref[...] = jnp.zeros_like(h2_ref)

    gi_ref[...] = (jnp.dot(xt_ref[...], wih0_ref[...],
                           preferred_element_type=jnp.float32)
                   + bih0_ref[...])

    def step(t, carry):
        h1, h2 = carry
        m12 = (jnp.dot(h1, w12_ref[...], preferred_element_type=jnp.float32)
               + b12_ref[...])
        gh2 = (jnp.dot(h2, whh1_ref[...], preferred_element_type=jnp.float32)
               + bhh1_ref[...])
        gi1 = gi_ref[pl.ds(jnp.minimum(t, CHUNK - 1) * B, B), :]
        h1n = _cell(gi1, m12[:, :3 * H], h1)
        h2n = _cell(m12[:, 3 * H:], gh2, h2)
        h1 = jnp.where(t < CHUNK, h1n, h1)
        h2 = jnp.where(t >= 1, h2n, h2)
        out_ref[pl.ds(jnp.maximum(t - 1, 0) * B, B), :] = h2
        return (h1, h2)

    h1, h2 = jax.lax.fori_loop(0, CHUNK + 1, step, (h1_ref[...], h2_ref[...]))
    h1_ref[...] = h1
    h2_ref[...] = h2


def kernel(x, w_ih_l0, w_hh_l0, b_ih_l0, b_hh_l0,
           w_ih_l1, w_hh_l1, b_ih_l1, b_hh_l1):
    xt = jnp.swapaxes(x, 0, 1).reshape(T * B, D)

    w12 = jnp.concatenate([w_hh_l0.T, w_ih_l1.T], axis=1)
    b12 = jnp.concatenate([b_hh_l0, b_ih_l1]).reshape(1, -1)

    full2d = lambda shape: pl.BlockSpec(shape, lambda i: (0, 0))
    out2d = pl.pallas_call(
        _gru2_kernel,
        grid=(NCHUNK,),
        in_specs=[
            pl.BlockSpec((CHUNK * B, D), lambda i: (i, 0)),
            full2d((D, 3 * H)),
            full2d((H, 6 * H)),
            full2d((1, 6 * H)),
            full2d((H, 3 * H)),
            full2d((1, 3 * H)),
            full2d((1, 3 * H)),
        ],
        out_specs=pl.BlockSpec((CHUNK * B, H), lambda i: (i, 0)),
        out_shape=jax.ShapeDtypeStruct((T * B, H), jnp.float32),
        scratch_shapes=[
            pltpu.VMEM((CHUNK * B, 3 * H), jnp.float32),
            pltpu.VMEM((B, H), jnp.float32),
            pltpu.VMEM((B, H), jnp.float32),
        ],
        compiler_params=pltpu.CompilerParams(
            dimension_semantics=("arbitrary",),
        ),
    )(xt, w_ih_l0.T, w12, b12, w_hh_l1.T, b_hh_l1.reshape(1, -1),
      b_ih_l0.reshape(1, -1))

    return jnp.swapaxes(out2d.reshape(T, B, H), 0, 1)


# speedup vs baseline: 13.9921x; 1.0127x over previous
import jax
import jax.numpy as jnp
from jax.experimental import pallas as pl
from jax.experimental.pallas import tpu as pltpu

B, T, D, H = 8, 2048, 256, 256
CHUNK = 256
NCHUNK = T // CHUNK


def _cell(gi, gh, h):
    rz = jax.nn.sigmoid(gi[:, :2 * H] + gh[:, :2 * H])
    r = rz[:, :H]
    z = rz[:, H:]
    n = jnp.tanh(gi[:, 2 * H:] + r * gh[:, 2 * H:])
    return n + z * (h - n)


def _gru2_kernel(xt_ref, wih0_ref, w12_ref, b12_ref, whh1_ref, bhh1_ref,
                 bih0_ref, out_ref, gi_ref, h1_ref, h2_ref):
    @pl.when(pl.program_id(0) == 0)
    def _init():
        h1_ref[...] = jnp.zeros_like(h1_ref)
        h2_ref[...] = jnp.zeros_like(h2_ref)

    gi_ref[...] = (jnp.dot(xt_ref[...], wih0_ref[...],
                           preferred_element_type=jnp.float32)
                   + bih0_ref[...])

    def step(t, carry):
        h1, h2 = carry
        m12 = (jnp.dot(h1.astype(jnp.bfloat16), w12_ref[...],
                       preferred_element_type=jnp.float32)
               + b12_ref[...])
        gh2 = (jnp.dot(h2.astype(jnp.bfloat16), whh1_ref[...],
                       preferred_element_type=jnp.float32)
               + bhh1_ref[...])
        gi1 = gi_ref[pl.ds(jnp.minimum(t, CHUNK - 1) * B, B), :]
        h1n = _cell(gi1, m12[:, :3 * H], h1)
        h2n = _cell(m12[:, 3 * H:], gh2, h2)
        h1 = jnp.where(t < CHUNK, h1n, h1)
        h2 = jnp.where(t >= 1, h2n, h2)
        out_ref[pl.ds(jnp.maximum(t - 1, 0) * B, B), :] = h2
        return (h1, h2)

    h1, h2 = jax.lax.fori_loop(0, CHUNK + 1, step, (h1_ref[...], h2_ref[...]))
    h1_ref[...] = h1
    h2_ref[...] = h2


def kernel(x, w_ih_l0, w_hh_l0, b_ih_l0, b_hh_l0,
           w_ih_l1, w_hh_l1, b_ih_l1, b_hh_l1):
    xt = jnp.swapaxes(x, 0, 1).reshape(T * B, D)

    w12 = jnp.concatenate([w_hh_l0.T, w_ih_l1.T],
                          axis=1).astype(jnp.bfloat16)
    b12 = jnp.concatenate([b_hh_l0, b_ih_l1]).reshape(1, -1)
    whh1 = w_hh_l1.T.astype(jnp.bfloat16)

    full2d = lambda shape: pl.BlockSpec(shape, lambda i: (0, 0))
    out2d = pl.pallas_call(
        _gru2_kernel,
        grid=(NCHUNK,),
        in_specs=[
            pl.BlockSpec((CHUNK * B, D), lambda i: (i, 0)),
            full2d((D, 3 * H)),
            full2d((H, 6 * H)),
            full2d((1, 6 * H)),
            full2d((H, 3 * H)),
            full2d((1, 3 * H)),
            full2d((1, 3 * H)),
        ],
        out_specs=pl.BlockSpec((CHUNK * B, H), lambda i: (i, 0)),
        out_shape=jax.ShapeDtypeStruct((T * B, H), jnp.float32),
        scratch_shapes=[
            pltpu.VMEM((CHUNK * B, 3 * H), jnp.float32),
            pltpu.VMEM((B, H), jnp.float32),
            pltpu.VMEM((B, H), jnp.float32),
        ],
        compiler_params=pltpu.CompilerParams(
            dimension_semantics=("arbitrary",),
        ),
    )(xt, w_ih_l0.T, w12, b12, whh1, b_hh_l1.reshape(1, -1),
      b_ih_l0.reshape(1, -1))

    return jnp.swapaxes(out2d.reshape(T, B, H), 0, 1)


# speedup vs baseline: 15.6257x; 1.1168x over previous
import jax
import jax.numpy as jnp
from jax.experimental import pallas as pl
from jax.experimental.pallas import tpu as pltpu

B, T, D, H = 8, 2048, 256, 256
CHUNK = 256
NCHUNK = T // CHUNK


def _gru2_kernel(xt_ref, wih0_ref, whh0_ref, wih1_ref, whh1_ref,
                 cb0_ref, cb1_ref, bn0_ref, bn1_ref,
                 out_ref, gi1_ref, gi2_ref, h1buf_ref, h1_ref, h2_ref):
    c = pl.program_id(0)

    @pl.when(c == 0)
    def _init():
        h1_ref[...] = jnp.zeros_like(h1_ref)
        h2_ref[...] = jnp.zeros_like(h2_ref)

    gi2_ref[...] = (jnp.dot(h1buf_ref[...], wih1_ref[...],
                            preferred_element_type=jnp.float32)
                    + cb1_ref[...])

    gi1_ref[...] = (jnp.dot(xt_ref[...], wih0_ref[...],
                            preferred_element_type=jnp.float32)
                    + cb0_ref[...])

    run1 = c < NCHUNK
    run2 = c >= 1

    def cell(gi, gh, bn, h):
        rz = jax.nn.sigmoid(gi[:, :2 * H] + gh[:, :2 * H])
        n = jnp.tanh(gi[:, 2 * H:] + rz[:, :H] * (gh[:, 2 * H:] + bn))
        return n + rz[:, H:] * (h - n)

    def step(t, carry):
        h1, h2 = carry
        gh1 = jnp.dot(h1.astype(jnp.bfloat16), whh0_ref[...],
                      preferred_element_type=jnp.float32)
        gh2 = jnp.dot(h2.astype(jnp.bfloat16), whh1_ref[...],
                      preferred_element_type=jnp.float32)
        gi1 = gi1_ref[pl.ds(t * B, B), :]
        gi2 = gi2_ref[pl.ds(t * B, B), :]
        h1 = jnp.where(run1, cell(gi1, gh1, bn0_ref[...], h1), h1)
        h2 = jnp.where(run2, cell(gi2, gh2, bn1_ref[...], h2), h2)
        h1buf_ref[pl.ds(t * B, B), :] = h1
        out_ref[pl.ds(t * B, B), :] = h2
        return (h1, h2)

    h1, h2 = jax.lax.fori_loop(0, CHUNK, step, (h1_ref[...], h2_ref[...]))
    h1_ref[...] = h1
    h2_ref[...] = h2


def kernel(x, w_ih_l0, w_hh_l0, b_ih_l0, b_hh_l0,
           w_ih_l1, w_hh_l1, b_ih_l1, b_hh_l1):
    xt = jnp.swapaxes(x, 0, 1).reshape(T * B, D)

    cb0 = jnp.concatenate([(b_ih_l0[:2 * H] + b_hh_l0[:2 * H]),
                           b_ih_l0[2 * H:]]).reshape(1, -1)
    cb1 = jnp.concatenate([(b_ih_l1[:2 * H] + b_hh_l1[:2 * H]),
                           b_ih_l1[2 * H:]]).reshape(1, -1)
    bn0 = b_hh_l0[2 * H:].reshape(1, -1)
    bn1 = b_hh_l1[2 * H:].reshape(1, -1)

    full2d = lambda shape: pl.BlockSpec(shape, lambda i: (0, 0))
    out2d = pl.pallas_call(
        _gru2_kernel,
        grid=(NCHUNK + 1,),
        in_specs=[
            pl.BlockSpec((CHUNK * B, D),
                         lambda c: (jnp.minimum(c, NCHUNK - 1), 0)),
            full2d((D, 3 * H)),
            full2d((H, 3 * H)),
            full2d((H, 3 * H)),
            full2d((H, 3 * H)),
            full2d((1, 3 * H)),
            full2d((1, 3 * H)),
            full2d((1, H)),
            full2d((1, H)),
        ],
        out_specs=pl.BlockSpec((CHUNK * B, H),
                               lambda c: (jnp.maximum(c - 1, 0), 0)),
        out_shape=jax.ShapeDtypeStruct((T * B, H), jnp.float32),
        scratch_shapes=[
            pltpu.VMEM((CHUNK * B, 3 * H), jnp.float32),
            pltpu.VMEM((CHUNK * B, 3 * H), jnp.float32),
            pltpu.VMEM((CHUNK * B, H), jnp.float32),
            pltpu.VMEM((B, H), jnp.float32),
            pltpu.VMEM((B, H), jnp.float32),
        ],
        compiler_params=pltpu.CompilerParams(
            dimension_semantics=("arbitrary",),
        ),
    )(xt, w_ih_l0.T, w_hh_l0.T.astype(jnp.bfloat16),
      w_ih_l1.T, w_hh_l1.T.astype(jnp.bfloat16),
      cb0, cb1, bn0, bn1)

    return jnp.swapaxes(out2d.reshape(T, B, H), 0, 1)


# speedup vs baseline: 17.9871x; 1.1511x over previous
import jax
import jax.numpy as jnp
from jax.experimental import pallas as pl
from jax.experimental.pallas import tpu as pltpu

B, T, D, H = 8, 2048, 256, 256
CHUNK = 256
NCHUNK = T // CHUNK


def _gru2_kernel(xt_ref, wih0_ref, whh0_ref, wih1_ref, whh1_ref,
                 cb0_ref, cb1_ref, bn0_ref, bn1_ref,
                 out_ref, gi1_ref, gi2_ref, h1buf_ref, h1_ref, h2_ref):
    c = pl.program_id(0)

    @pl.when(c == 0)
    def _init():
        h1_ref[...] = jnp.zeros_like(h1_ref)
        h2_ref[...] = jnp.zeros_like(h2_ref)

    gi2_ref[...] = (jnp.dot(h1buf_ref[...], wih1_ref[...],
                            preferred_element_type=jnp.float32)
                    + cb1_ref[...])

    gi1_ref[...] = (jnp.dot(xt_ref[...], wih0_ref[...],
                            preferred_element_type=jnp.float32)
                    + cb0_ref[...])

    run1 = c < NCHUNK
    run2 = c >= 1

    def cell(gi, gh, bn, h):
        rz = jax.nn.sigmoid(gi[:, :2 * H] + gh[:, :2 * H])
        n = jnp.tanh(gi[:, 2 * H:] + rz[:, :H] * (gh[:, 2 * H:] + bn))
        return n + rz[:, H:] * (h - n)

    def substep(t, h1, h2):
        gh1 = jnp.dot(h1.astype(jnp.bfloat16), whh0_ref[...],
                      preferred_element_type=jnp.float32)
        gh2 = jnp.dot(h2.astype(jnp.bfloat16), whh1_ref[...],
                      preferred_element_type=jnp.float32)
        gi1 = gi1_ref[pl.ds(t * B, B), :]
        gi2 = gi2_ref[pl.ds(t * B, B), :]
        h1 = jnp.where(run1, cell(gi1, gh1, bn0_ref[...], h1), h1)
        h2 = jnp.where(run2, cell(gi2, gh2, bn1_ref[...], h2), h2)
        h1buf_ref[pl.ds(t * B, B), :] = h1
        out_ref[pl.ds(t * B, B), :] = h2
        return h1, h2

    def step(i, carry):
        h1, h2 = carry
        h1, h2 = substep(2 * i, h1, h2)
        h1, h2 = substep(2 * i + 1, h1, h2)
        return (h1, h2)

    h1, h2 = jax.lax.fori_loop(0, CHUNK // 2, step, (h1_ref[...], h2_ref[...]))
    h1_ref[...] = h1
    h2_ref[...] = h2


def kernel(x, w_ih_l0, w_hh_l0, b_ih_l0, b_hh_l0,
           w_ih_l1, w_hh_l1, b_ih_l1, b_hh_l1):
    xt = jnp.swapaxes(x, 0, 1).reshape(T * B, D)

    cb0 = jnp.concatenate([(b_ih_l0[:2 * H] + b_hh_l0[:2 * H]),
                           b_ih_l0[2 * H:]]).reshape(1, -1)
    cb1 = jnp.concatenate([(b_ih_l1[:2 * H] + b_hh_l1[:2 * H]),
                           b_ih_l1[2 * H:]]).reshape(1, -1)
    bn0 = b_hh_l0[2 * H:].reshape(1, -1)
    bn1 = b_hh_l1[2 * H:].reshape(1, -1)

    full2d = lambda shape: pl.BlockSpec(shape, lambda i: (0, 0))
    out2d = pl.pallas_call(
        _gru2_kernel,
        grid=(NCHUNK + 1,),
        in_specs=[
            pl.BlockSpec((CHUNK * B, D),
                         lambda c: (jnp.minimum(c, NCHUNK - 1), 0)),
            full2d((D, 3 * H)),
            full2d((H, 3 * H)),
            full2d((H, 3 * H)),
            full2d((H, 3 * H)),
            full2d((1, 3 * H)),
            full2d((1, 3 * H)),
            full2d((1, H)),
            full2d((1, H)),
        ],
        out_specs=pl.BlockSpec((CHUNK * B, H),
                               lambda c: (jnp.maximum(c - 1, 0), 0)),
        out_shape=jax.ShapeDtypeStruct((T * B, H), jnp.float32),
        scratch_shapes=[
            pltpu.VMEM((CHUNK * B, 3 * H), jnp.float32),
            pltpu.VMEM((CHUNK * B, 3 * H), jnp.float32),
            pltpu.VMEM((CHUNK * B, H), jnp.float32),
            pltpu.VMEM((B, H), jnp.float32),
            pltpu.VMEM((B, H), jnp.float32),
        ],
        compiler_params=pltpu.CompilerParams(
            dimension_semantics=("arbitrary",),
        ),
    )(xt, w_ih_l0.T, w_hh_l0.T.astype(jnp.bfloat16),
      w_ih_l1.T, w_hh_l1.T.astype(jnp.bfloat16),
      cb0, cb1, bn0, bn1)

    return jnp.swapaxes(out2d.reshape(T, B, H), 0, 1)


# speedup vs baseline: 19.5093x; 1.0846x over previous
import jax
import jax.numpy as jnp
from jax.experimental import pallas as pl
from jax.experimental.pallas import tpu as pltpu

B, T, D, H = 8, 2048, 256, 256
CHUNK = 256
NCHUNK = T // CHUNK


def _gru2_kernel(xt_ref, wih0_ref, whh0_ref, wih1_ref, whh1_ref,
                 cb0_ref, cb1_ref, bn0_ref, bn1_ref,
                 out_ref, gi1_ref, gi2_ref, h1buf_ref, h1_ref, h2_ref):
    c = pl.program_id(0)

    @pl.when(c == 0)
    def _init():
        h1_ref[...] = jnp.zeros_like(h1_ref)
        h2_ref[...] = jnp.zeros_like(h2_ref)

    gi2_ref[...] = (jnp.dot(h1buf_ref[...], wih1_ref[...],
                            preferred_element_type=jnp.float32)
                    + cb1_ref[...])

    gi1_ref[...] = (jnp.dot(xt_ref[...], wih0_ref[...],
                            preferred_element_type=jnp.float32)
                    + cb0_ref[...])

    run1 = c < NCHUNK
    run2 = c >= 1

    def cell(gi, gh, bn, h):
        rz = jax.nn.sigmoid(gi[:, :2 * H] + gh[:, :2 * H])
        n = jnp.tanh(gi[:, 2 * H:] + rz[:, :H] * (gh[:, 2 * H:] + bn))
        return n + rz[:, H:] * (h - n)

    def substep(t, h1, h2):
        gh1 = jnp.dot(h1.astype(jnp.bfloat16), whh0_ref[...],
                      preferred_element_type=jnp.float32)
        gh2 = jnp.dot(h2.astype(jnp.bfloat16), whh1_ref[...],
                      preferred_element_type=jnp.float32)
        gi1 = gi1_ref[pl.ds(t * B, B), :]
        gi2 = gi2_ref[pl.ds(t * B, B), :]
        h1 = jnp.where(run1, cell(gi1, gh1, bn0_ref[...], h1), h1)
        h2 = jnp.where(run2, cell(gi2, gh2, bn1_ref[...], h2), h2)
        h1buf_ref[pl.ds(t * B, B), :] = h1
        out_ref[pl.ds(t * B, B), :] = h2
        return h1, h2

    def step(i, carry):
        h1, h2 = carry
        h1, h2 = substep(4 * i, h1, h2)
        h1, h2 = substep(4 * i + 1, h1, h2)
        h1, h2 = substep(4 * i + 2, h1, h2)
        h1, h2 = substep(4 * i + 3, h1, h2)
        return (h1, h2)

    h1, h2 = jax.lax.fori_loop(0, CHUNK // 4, step, (h1_ref[...], h2_ref[...]))
    h1_ref[...] = h1
    h2_ref[...] = h2


def kernel(x, w_ih_l0, w_hh_l0, b_ih_l0, b_hh_l0,
           w_ih_l1, w_hh_l1, b_ih_l1, b_hh_l1):
    xt = jnp.swapaxes(x, 0, 1).reshape(T * B, D)

    cb0 = jnp.concatenate([(b_ih_l0[:2 * H] + b_hh_l0[:2 * H]),
                           b_ih_l0[2 * H:]]).reshape(1, -1)
    cb1 = jnp.concatenate([(b_ih_l1[:2 * H] + b_hh_l1[:2 * H]),
                           b_ih_l1[2 * H:]]).reshape(1, -1)
    bn0 = b_hh_l0[2 * H:].reshape(1, -1)
    bn1 = b_hh_l1[2 * H:].reshape(1, -1)

    full2d = lambda shape: pl.BlockSpec(shape, lambda i: (0, 0))
    out2d = pl.pallas_call(
        _gru2_kernel,
        grid=(NCHUNK + 1,),
        in_specs=[
            pl.BlockSpec((CHUNK * B, D),
                         lambda c: (jnp.minimum(c, NCHUNK - 1), 0)),
            full2d((D, 3 * H)),
            full2d((H, 3 * H)),
            full2d((H, 3 * H)),
            full2d((H, 3 * H)),
            full2d((1, 3 * H)),
            full2d((1, 3 * H)),
            full2d((1, H)),
            full2d((1, H)),
        ],
        out_specs=pl.BlockSpec((CHUNK * B, H),
                               lambda c: (jnp.maximum(c - 1, 0), 0)),
        out_shape=jax.ShapeDtypeStruct((T * B, H), jnp.float32),
        scratch_shapes=[
            pltpu.VMEM((CHUNK * B, 3 * H), jnp.float32),
            pltpu.VMEM((CHUNK * B, 3 * H), jnp.float32),
            pltpu.VMEM((CHUNK * B, H), jnp.float32),
            pltpu.VMEM((B, H), jnp.float32),
            pltpu.VMEM((B, H), jnp.float32),
        ],
        compiler_params=pltpu.CompilerParams(
            dimension_semantics=("arbitrary",),
        ),
    )(xt, w_ih_l0.T, w_hh_l0.T.astype(jnp.bfloat16),
      w_ih_l1.T, w_hh_l1.T.astype(jnp.bfloat16),
      cb0, cb1, bn0, bn1)

    return jnp.swapaxes(out2d.reshape(T, B, H), 0, 1)


# speedup vs baseline: 20.3890x; 1.0451x over previous
import jax
import jax.numpy as jnp
from jax.experimental import pallas as pl
from jax.experimental.pallas import tpu as pltpu

B, T, D, H = 8, 2048, 256, 256
CHUNK = 256
NCHUNK = T // CHUNK


def _gru2_kernel(xt_ref, wih0_ref, whh0_ref, wih1_ref, whh1_ref,
                 cb0_ref, cb1_ref, bn0_ref, bn1_ref,
                 out_ref, gi1_ref, gi2_ref, h1buf_ref, h1_ref, h2_ref):
    c = pl.program_id(0)

    @pl.when(c == 0)
    def _init():
        h1_ref[...] = jnp.zeros_like(h1_ref)
        h2_ref[...] = jnp.zeros_like(h2_ref)

    gi2_ref[...] = (jnp.dot(h1buf_ref[...], wih1_ref[...],
                            preferred_element_type=jnp.float32)
                    + cb1_ref[...])

    gi1_ref[...] = (jnp.dot(xt_ref[...], wih0_ref[...],
                            preferred_element_type=jnp.float32)
                    + cb0_ref[...])

    run1 = c < NCHUNK
    run2 = c >= 1

    def cell(gi, gh, bn, h):
        rz = jax.nn.sigmoid(gi[:, :2 * H] + gh[:, :2 * H])
        n = jnp.tanh(gi[:, 2 * H:] + rz[:, :H] * (gh[:, 2 * H:] + bn))
        return n + rz[:, H:] * (h - n)

    def substep(t, h1, h2):
        gh1 = jnp.dot(h1.astype(jnp.bfloat16), whh0_ref[...],
                      preferred_element_type=jnp.float32)
        gh2 = jnp.dot(h2.astype(jnp.bfloat16), whh1_ref[...],
                      preferred_element_type=jnp.float32)
        gi1 = gi1_ref[pl.ds(t * B, B), :]
        gi2 = gi2_ref[pl.ds(t * B, B), :]
        h1 = jnp.where(run1, cell(gi1, gh1, bn0_ref[...], h1), h1)
        h2 = jnp.where(run2, cell(gi2, gh2, bn1_ref[...], h2), h2)
        h1buf_ref[pl.ds(t * B, B), :] = h1
        out_ref[pl.ds(t * B, B), :] = h2
        return h1, h2

    def step(i, carry):
        h1, h2 = carry
        for k in range(8):
            h1, h2 = substep(8 * i + k, h1, h2)
        return (h1, h2)

    h1, h2 = jax.lax.fori_loop(0, CHUNK // 8, step, (h1_ref[...], h2_ref[...]))
    h1_ref[...] = h1
    h2_ref[...] = h2


def kernel(x, w_ih_l0, w_hh_l0, b_ih_l0, b_hh_l0,
           w_ih_l1, w_hh_l1, b_ih_l1, b_hh_l1):
    xt = jnp.swapaxes(x, 0, 1).reshape(T * B, D)

    cb0 = jnp.concatenate([(b_ih_l0[:2 * H] + b_hh_l0[:2 * H]),
                           b_ih_l0[2 * H:]]).reshape(1, -1)
    cb1 = jnp.concatenate([(b_ih_l1[:2 * H] + b_hh_l1[:2 * H]),
                           b_ih_l1[2 * H:]]).reshape(1, -1)
    bn0 = b_hh_l0[2 * H:].reshape(1, -1)
    bn1 = b_hh_l1[2 * H:].reshape(1, -1)

    full2d = lambda shape: pl.BlockSpec(shape, lambda i: (0, 0))
    out2d = pl.pallas_call(
        _gru2_kernel,
        grid=(NCHUNK + 1,),
        in_specs=[
            pl.BlockSpec((CHUNK * B, D),
                         lambda c: (jnp.minimum(c, NCHUNK - 1), 0)),
            full2d((D, 3 * H)),
            full2d((H, 3 * H)),
            full2d((H, 3 * H)),
            full2d((H, 3 * H)),
            full2d((1, 3 * H)),
            full2d((1, 3 * H)),
            full2d((1, H)),
            full2d((1, H)),
        ],
        out_specs=pl.BlockSpec((CHUNK * B, H),
                               lambda c: (jnp.maximum(c - 1, 0), 0)),
        out_shape=jax.ShapeDtypeStruct((T * B, H), jnp.float32),
        scratch_shapes=[
            pltpu.VMEM((CHUNK * B, 3 * H), jnp.float32),
            pltpu.VMEM((CHUNK * B, 3 * H), jnp.float32),
            pltpu.VMEM((CHUNK * B, H), jnp.float32),
            pltpu.VMEM((B, H), jnp.float32),
            pltpu.VMEM((B, H), jnp.float32),
        ],
        compiler_params=pltpu.CompilerParams(
            dimension_semantics=("arbitrary",),
        ),
    )(xt, w_ih_l0.T, w_hh_l0.T.astype(jnp.bfloat16),
      w_ih_l1.T, w_hh_l1.T.astype(jnp.bfloat16),
      cb0, cb1, bn0, bn1)

    return jnp.swapaxes(out2d.reshape(T, B, H), 0, 1)


# speedup vs baseline: 20.8728x; 1.0237x over previous
import jax
import jax.numpy as jnp
from jax.experimental import pallas as pl
from jax.experimental.pallas import tpu as pltpu

B, T, D, H = 8, 2048, 256, 256
CHUNK = 256
NCHUNK = T // CHUNK


def _gru2_kernel(xt_ref, wih0_ref, whh0_ref, wih1_ref, whh1_ref,
                 cb0_ref, cb1_ref, bn0_ref, bn1_ref,
                 out_ref, gi1_ref, gi2_ref, h1buf_ref, h1_ref, h2_ref):
    c = pl.program_id(0)

    @pl.when(c == 0)
    def _init():
        h1_ref[...] = jnp.zeros_like(h1_ref)
        h2_ref[...] = jnp.zeros_like(h2_ref)

    gi2_ref[...] = (jnp.dot(h1buf_ref[...], wih1_ref[...],
                            preferred_element_type=jnp.float32)
                    + cb1_ref[...])

    gi1_ref[...] = (jnp.dot(xt_ref[...], wih0_ref[...],
                            preferred_element_type=jnp.float32)
                    + cb0_ref[...])

    run1 = c < NCHUNK
    run2 = c >= 1

    def cell(gi, gh, bn, h):
        rz = jax.nn.sigmoid(gi[:, :2 * H] + gh[:, :2 * H])
        n = jnp.tanh(gi[:, 2 * H:] + rz[:, :H] * (gh[:, 2 * H:] + bn))
        return n + rz[:, H:] * (h - n)

    def substep(t, h1, h2):
        gh1 = jnp.dot(h1.astype(jnp.bfloat16), whh0_ref[...],
                      preferred_element_type=jnp.float32)
        gh2 = jnp.dot(h2.astype(jnp.bfloat16), whh1_ref[...],
                      preferred_element_type=jnp.float32)
        gi1 = gi1_ref[pl.ds(t * B, B), :]
        gi2 = gi2_ref[pl.ds(t * B, B), :]
        h1 = jnp.where(run1, cell(gi1, gh1, bn0_ref[...], h1), h1)
        h2 = jnp.where(run2, cell(gi2, gh2, bn1_ref[...], h2), h2)
        h1buf_ref[pl.ds(t * B, B), :] = h1
        out_ref[pl.ds(t * B, B), :] = h2
        return h1, h2

    def step(i, carry):
        h1, h2 = carry
        for k in range(16):
            h1, h2 = substep(16 * i + k, h1, h2)
        return (h1, h2)

    h1, h2 = jax.lax.fori_loop(0, CHUNK // 16, step, (h1_ref[...], h2_ref[...]))
    h1_ref[...] = h1
    h2_ref[...] = h2


def kernel(x, w_ih_l0, w_hh_l0, b_ih_l0, b_hh_l0,
           w_ih_l1, w_hh_l1, b_ih_l1, b_hh_l1):
    xt = jnp.swapaxes(x, 0, 1).reshape(T * B, D)

    cb0 = jnp.concatenate([(b_ih_l0[:2 * H] + b_hh_l0[:2 * H]),
                           b_ih_l0[2 * H:]]).reshape(1, -1)
    cb1 = jnp.concatenate([(b_ih_l1[:2 * H] + b_hh_l1[:2 * H]),
                           b_ih_l1[2 * H:]]).reshape(1, -1)
    bn0 = b_hh_l0[2 * H:].reshape(1, -1)
    bn1 = b_hh_l1[2 * H:].reshape(1, -1)

    full2d = lambda shape: pl.BlockSpec(shape, lambda i: (0, 0))
    out2d = pl.pallas_call(
        _gru2_kernel,
        grid=(NCHUNK + 1,),
        in_specs=[
            pl.BlockSpec((CHUNK * B, D),
                         lambda c: (jnp.minimum(c, NCHUNK - 1), 0)),
            full2d((D, 3 * H)),
            full2d((H, 3 * H)),
            full2d((H, 3 * H)),
            full2d((H, 3 * H)),
            full2d((1, 3 * H)),
            full2d((1, 3 * H)),
            full2d((1, H)),
            full2d((1, H)),
        ],
        out_specs=pl.BlockSpec((CHUNK * B, H),
                               lambda c: (jnp.maximum(c - 1, 0), 0)),
        out_shape=jax.ShapeDtypeStruct((T * B, H), jnp.float32),
        scratch_shapes=[
            pltpu.VMEM((CHUNK * B, 3 * H), jnp.float32),
            pltpu.VMEM((CHUNK * B, 3 * H), jnp.float32),
            pltpu.VMEM((CHUNK * B, H), jnp.float32),
            pltpu.VMEM((B, H), jnp.float32),
            pltpu.VMEM((B, H), jnp.float32),
        ],
        compiler_params=pltpu.CompilerParams(
            dimension_semantics=("arbitrary",),
        ),
    )(xt, w_ih_l0.T, w_hh_l0.T.astype(jnp.bfloat16),
      w_ih_l1.T, w_hh_l1.T.astype(jnp.bfloat16),
      cb0, cb1, bn0, bn1)

    return jnp.swapaxes(out2d.reshape(T, B, H), 0, 1)


# speedup vs baseline: 21.0184x; 1.0070x over previous
import jax
import jax.numpy as jnp
from jax.experimental import pallas as pl
from jax.experimental.pallas import tpu as pltpu

B, T, D, H = 8, 2048, 256, 256
CHUNK = 256
NCHUNK = T // CHUNK


def _gru2_kernel(xt_ref, wih0_ref, whh0_ref, wih1_ref, whh1_ref,
                 cb0_ref, cb1_ref, bn0_ref, bn1_ref,
                 out_ref, gi1_ref, gi2_ref, h1buf_ref, h1_ref, h2_ref):
    c = pl.program_id(0)

    @pl.when(c == 0)
    def _init():
        h1_ref[...] = jnp.zeros_like(h1_ref)
        h2_ref[...] = jnp.zeros_like(h2_ref)

    @pl.when(c >= 1)
    def _gi2():
        gi2_ref[...] = (jnp.dot(h1buf_ref[...], wih1_ref[...],
                                preferred_element_type=jnp.float32)
                        + cb1_ref[...])

    @pl.when(c < NCHUNK)
    def _gi1():
        gi1_ref[...] = (jnp.dot(xt_ref[...], wih0_ref[...],
                                preferred_element_type=jnp.float32)
                        + cb0_ref[...])

    def cell(gi, gh, bn, h):
        rz = jax.nn.sigmoid(gi[:, :2 * H] + gh[:, :2 * H])
        n = jnp.tanh(gi[:, 2 * H:] + rz[:, :H] * (gh[:, 2 * H:] + bn))
        return n + rz[:, H:] * (h - n)

    def sub1(t, h1):
        gh1 = jnp.dot(h1.astype(jnp.bfloat16), whh0_ref[...],
                      preferred_element_type=jnp.float32)
        h1 = cell(gi1_ref[pl.ds(t * B, B), :], gh1, bn0_ref[...], h1)
        h1buf_ref[pl.ds(t * B, B), :] = h1
        return h1

    def sub2(t, h2):
        gh2 = jnp.dot(h2.astype(jnp.bfloat16), whh1_ref[...],
                      preferred_element_type=jnp.float32)
        h2 = cell(gi2_ref[pl.ds(t * B, B), :], gh2, bn1_ref[...], h2)
        out_ref[pl.ds(t * B, B), :] = h2
        return h2

    @pl.when(c == 0)
    def _first():
        def step(i, h1):
            for k in range(16):
                h1 = sub1(16 * i + k, h1)
            return h1
        h1_ref[...] = jax.lax.fori_loop(0, CHUNK // 16, step, h1_ref[...])

    @pl.when(jnp.logical_and(c >= 1, c < NCHUNK))
    def _main():
        def step(i, carry):
            h1, h2 = carry
            for k in range(16):
                t = 16 * i + k
                h1 = sub1(t, h1)
                h2 = sub2(t, h2)
            return (h1, h2)
        h1, h2 = jax.lax.fori_loop(0, CHUNK // 16, step,
                                   (h1_ref[...], h2_ref[...]))
        h1_ref[...] = h1
        h2_ref[...] = h2

    @pl.when(c == NCHUNK)
    def _drain():
        def step(i, h2):
            for k in range(16):
                h2 = sub2(16 * i + k, h2)
            return h2
        h2_ref[...] = jax.lax.fori_loop(0, CHUNK // 16, step, h2_ref[...])


def kernel(x, w_ih_l0, w_hh_l0, b_ih_l0, b_hh_l0,
           w_ih_l1, w_hh_l1, b_ih_l1, b_hh_l1):
    xt = jnp.swapaxes(x, 0, 1).reshape(T * B, D)

    cb0 = jnp.concatenate([(b_ih_l0[:2 * H] + b_hh_l0[:2 * H]),
                           b_ih_l0[2 * H:]]).reshape(1, -1)
    cb1 = jnp.concatenate([(b_ih_l1[:2 * H] + b_hh_l1[:2 * H]),
                           b_ih_l1[2 * H:]]).reshape(1, -1)
    bn0 = b_hh_l0[2 * H:].reshape(1, -1)
    bn1 = b_hh_l1[2 * H:].reshape(1, -1)

    full2d = lambda shape: pl.BlockSpec(shape, lambda i: (0, 0))
    out2d = pl.pallas_call(
        _gru2_kernel,
        grid=(NCHUNK + 1,),
        in_specs=[
            pl.BlockSpec((CHUNK * B, D),
                         lambda c: (jnp.minimum(c, NCHUNK - 1), 0)),
            full2d((D, 3 * H)),
            full2d((H, 3 * H)),
            full2d((H, 3 * H)),
            full2d((H, 3 * H)),
            full2d((1, 3 * H)),
            full2d((1, 3 * H)),
            full2d((1, H)),
            full2d((1, H)),
        ],
        out_specs=pl.BlockSpec((CHUNK * B, H),
                               lambda c: (jnp.maximum(c - 1, 0), 0)),
        out_shape=jax.ShapeDtypeStruct((T * B, H), jnp.float32),
        scratch_shapes=[
            pltpu.VMEM((CHUNK * B, 3 * H), jnp.float32),
            pltpu.VMEM((CHUNK * B, 3 * H), jnp.float32),
            pltpu.VMEM((CHUNK * B, H), jnp.float32),
            pltpu.VMEM((B, H), jnp.float32),
            pltpu.VMEM((B, H), jnp.float32),
        ],
        compiler_params=pltpu.CompilerParams(
            dimension_semantics=("arbitrary",),
        ),
    )(xt, w_ih_l0.T, w_hh_l0.T.astype(jnp.bfloat16),
      w_ih_l1.T, w_hh_l1.T.astype(jnp.bfloat16),
      cb0, cb1, bn0, bn1)

    return jnp.swapaxes(out2d.reshape(T, B, H), 0, 1)
